# Initial kernel scaffold; baseline (speedup 1.0000x reference)
import jax
import jax.numpy as jnp
from jax.experimental import pallas as pl


def kernel(data_x, data_edge_index, W1, b1, W2, b2, W3, b3, Wr, br):
    raise NotImplementedError("write your pallas kernel here")


# speedup vs baseline: 16.0803x; 16.0803x over previous
import functools

import jax
import jax.numpy as jnp
from jax import lax
from jax.experimental import pallas as pl
from jax.experimental.pallas import tpu as pltpu
from jax.experimental.pallas import tpu_sc as plsc

NC = 2
NS = 16
NW = NC * NS
BLK = 128
CH = 8


def _pad_blocks(e):
    per_w = -(-e // (NW * BLK * CH)) * (BLK * CH)
    return per_w * NW


def _pad_nodes(n):
    return -(-n // (NS * 8)) * (NS * 8)


@functools.lru_cache(maxsize=None)
def _deg_kernel(n, e_pad):
    epw = e_pad // NW
    rounds = epw // (BLK * CH)
    kpw = epw // BLK
    rpt = n // NS
    mesh = plsc.VectorSubcoreMesh(core_axis_name="c", subcore_axis_name="s",
                                  num_cores=NC, num_subcores=NS)

    def body(src3, dst3, zz, mtab, out, acc, sbufs, dbufs, mibuf, mrows, sem):
        c = lax.axis_index("c")
        s = lax.axis_index("s")
        wid = c * NS + s
        r0 = s * rpt
        pltpu.sync_copy(zz.at[pl.ds(r0, rpt)], acc.at[pl.ds(r0, rpt)])
        plsc.subcore_barrier()

        def round_body(r, _):
            boff = wid * kpw + r * CH
            pltpu.sync_copy(src3.at[pl.ds(boff, CH)], sbufs)
            pltpu.sync_copy(dst3.at[pl.ds(boff, CH)], dbufs)
            for j in range(CH):
                for k in range(BLK // 16):
                    s16 = sbufs[j, pl.ds(k * 16, 16)]
                    d16 = dbufs[j, pl.ds(k * 16, 16)]
                    mibuf[j, pl.ds(k * 16, 16)] = jnp.where(
                        s16 != d16, 1, 0).astype(jnp.int32)
            descs = []
            for j in range(CH):
                descs.append(pltpu.async_copy(mtab.at[mibuf.at[j]],
                                              mrows.at[j], sem))
            for d in descs:
                d.wait()
            for j in range(CH):
                pltpu.sync_copy(mrows.at[j], acc.at[sbufs.at[j]], add=True)
                pltpu.sync_copy(mrows.at[j], acc.at[dbufs.at[j]], add=True)
            return 0

        lax.fori_loop(0, rounds, round_body, 0)
        plsc.subcore_barrier()
        pltpu.sync_copy(acc.at[pl.ds(r0, rpt)], out.at[c, pl.ds(r0, rpt)])

    return pl.kernel(
        body,
        out_type=jax.ShapeDtypeStruct((NC, n, 16), jnp.float32),
        mesh=mesh,
        compiler_params=pltpu.CompilerParams(use_tc_tiling_on_sc=False),
        scratch_types=[
            pltpu.VMEM_SHARED((n, 16), jnp.float32),
            pltpu.VMEM((CH, BLK), jnp.int32),
            pltpu.VMEM((CH, BLK), jnp.int32),
            pltpu.VMEM((CH, BLK), jnp.int32),
            pltpu.VMEM((CH, BLK, 16), jnp.float32),
            pltpu.SemaphoreType.DMA,
        ],
    )


@functools.lru_cache(maxsize=None)
def _prop_kernel(n, e_pad, f):
    epw = e_pad // NW
    rounds = epw // (BLK * CH)
    kpw = epw // BLK
    rpt = n // NS
    mesh = plsc.VectorSubcoreMesh(core_axis_name="c", subcore_axis_name="s",
                                  num_cores=NC, num_subcores=NS)

    def body(g, src3, dst3, zz, plus_o, minus_o,
             plus_s, minus_s, sbufs, dbufs, rows_a, rows_b, sem_a, sem_b):
        c = lax.axis_index("c")
        s = lax.axis_index("s")
        wid = c * NS + s
        r0 = s * rpt
        pltpu.sync_copy(zz.at[pl.ds(r0, rpt)], plus_s.at[pl.ds(r0, rpt)])
        pltpu.sync_copy(zz.at[pl.ds(r0, rpt)], minus_s.at[pl.ds(r0, rpt)])
        plsc.subcore_barrier()

        def round_body(r, _):
            boff = wid * kpw + r * CH
            pltpu.sync_copy(src3.at[pl.ds(boff, CH)], sbufs)
            pltpu.sync_copy(dst3.at[pl.ds(boff, CH)], dbufs)
            descs = []
            for j in range(CH):
                descs.append(pltpu.async_copy(g.at[dbufs.at[j]],
                                              rows_a.at[j], sem_a))
                descs.append(pltpu.async_copy(g.at[sbufs.at[j]],
                                              rows_b.at[j], sem_b))
            for d in descs:
                d.wait()
            for j in range(CH):
                pltpu.sync_copy(rows_a.at[j], plus_s.at[sbufs.at[j]],
                                add=True)
                pltpu.sync_copy(rows_b.at[j], minus_s.at[dbufs.at[j]],
                                add=True)
            return 0

        lax.fori_loop(0, rounds, round_body, 0)
        plsc.subcore_barrier()
        pltpu.sync_copy(plus_s.at[pl.ds(r0, rpt)],
                        plus_o.at[c, pl.ds(r0, rpt)])
        pltpu.sync_copy(minus_s.at[pl.ds(r0, rpt)],
                        minus_o.at[c, pl.ds(r0, rpt)])

    return pl.kernel(
        body,
        out_type=[jax.ShapeDtypeStruct((NC, n, f), jnp.float32),
                  jax.ShapeDtypeStruct((NC, n, f), jnp.float32)],
        mesh=mesh,
        compiler_params=pltpu.CompilerParams(use_tc_tiling_on_sc=False),
        scratch_types=[
            pltpu.VMEM_SHARED((n, f), jnp.float32),
            pltpu.VMEM_SHARED((n, f), jnp.float32),
            pltpu.VMEM((CH, BLK), jnp.int32),
            pltpu.VMEM((CH, BLK), jnp.int32),
            pltpu.VMEM((CH, BLK, f), jnp.float32),
            pltpu.VMEM((CH, BLK, f), jnp.float32),
            pltpu.SemaphoreType.DMA,
            pltpu.SemaphoreType.DMA,
        ],
    )


def _t1_body(x_ref, w_ref, b_ref, degacc_ref, y0b_ref, g1_ref, dinv_ref):
    y = jnp.dot(x_ref[...], w_ref[...], preferred_element_type=jnp.float32)
    dacc = degacc_ref[...]
    deg = dacc[0, :, 0] + dacc[1, :, 0]
    dinv = jnp.where(deg > 0, lax.rsqrt(deg), 0.0)[:, None]
    y0b_ref[...] = y[:, :16] + b_ref[...]
    g1_ref[...] = dinv * y[:, 16:32]
    dinv_ref[...] = dinv


def _t2_body(y0b_ref, p_ref, m_ref, dinv_ref, w_ref, b_ref,
             or_ref, oi_ref, g2_ref):
    dinv = dinv_ref[...]
    p = p_ref[...]
    m = m_ref[...]
    u = (0.5 * dinv) * (p[0] + p[1] - m[0] - m[1])
    y0b = y0b_ref[...]
    xr = jnp.maximum(y0b - u, 0.0)
    xi = y0b + u
    zr = jnp.dot(xr, w_ref[...], preferred_element_type=jnp.float32)
    zi = jnp.dot(xi, w_ref[...], preferred_element_type=jnp.float32)
    b = b_ref[...]
    or_ref[...] = zr[:, :16] + b
    oi_ref[...] = zi[:, :16] + b
    g2_ref[...] = dinv * jnp.concatenate([zr[:, 16:], zi[:, 16:]], axis=1)


def _t3_body(or_ref, oi_ref, p_ref, m_ref, dinv_ref, w_ref, b_ref,
             or3_ref, oi3_ref, g3_ref):
    dinv = dinv_ref[...]
    p = p_ref[...]
    m = m_ref[...]
    u = (0.5 * dinv) * (p[0] + p[1] - m[0] - m[1])
    xr = jnp.maximum(or_ref[...] - u[:, 16:32], 0.0)
    xi = oi_ref[...] + u[:, 0:16]
    zr = jnp.dot(xr, w_ref[...], preferred_element_type=jnp.float32)
    zi = jnp.dot(xi, w_ref[...], preferred_element_type=jnp.float32)
    b = b_ref[...]
    or3_ref[...] = zr[:, :8] + b
    oi3_ref[...] = zi[:, :8] + b
    g3_ref[...] = dinv * jnp.concatenate([zr[:, 8:], zi[:, 8:]], axis=1)


def _t4_body(or3_ref, oi3_ref, p_ref, m_ref, dinv_ref, wr_ref, br_ref,
             out_ref):
    dinv = dinv_ref[...]
    p = p_ref[...]
    m = m_ref[...]
    u = (0.5 * dinv) * (p[0] + p[1] - m[0] - m[1])
    xr = or3_ref[...] - u[:, 8:16]
    xi = oi3_ref[...] + u[:, 0:8]
    h = jnp.concatenate([xr, xi], axis=1)
    out_ref[...] = jnp.dot(h, wr_ref[...],
                           preferred_element_type=jnp.float32) + br_ref[...]


def _tc_call(body, out_shapes, *args):
    return pl.pallas_call(body, out_shape=out_shapes)(*args)


def kernel(data_x, data_edge_index, W1, b1, W2, b2, W3, b3, Wr, br):
    n, _ = data_x.shape
    e = data_edge_index.shape[1]
    e_pad = _pad_blocks(e)
    n_pad = _pad_nodes(n)

    src = data_edge_index[0]
    dst = data_edge_index[1]
    pad = e_pad - e
    zpad = jnp.zeros((pad,), jnp.int32)
    src3 = jnp.concatenate([src, zpad]).reshape(e_pad // BLK, BLK)
    dst3 = jnp.concatenate([dst, zpad]).reshape(e_pad // BLK, BLK)

    def rowpad(a):
        return jnp.concatenate(
            [a, jnp.zeros((n_pad - n, a.shape[1]), a.dtype)])

    zz16 = jnp.zeros((n_pad, 16), jnp.float32)
    zz32 = jnp.zeros((n_pad, 32), jnp.float32)
    mtab = jnp.zeros((8, 16), jnp.float32).at[1, 0].set(0.5)

    w1cat = jnp.concatenate([W1[0], W1[1]], axis=1)
    w2cat = jnp.concatenate([W2[0], W2[1]], axis=1)
    w3cat = jnp.concatenate([W3[0], W3[1]], axis=1)
    b1r = b1.reshape(1, -1)
    b2r = b2.reshape(1, -1)
    b3r = b3.reshape(1, -1)
    brr = br.reshape(1, -1)

    f32 = jnp.float32
    degacc = _deg_kernel(n_pad, e_pad)(src3, dst3, zz16, mtab)[:, :n]

    y0b, g1, dinv = _tc_call(
        _t1_body,
        [jax.ShapeDtypeStruct((n, 16), f32),
         jax.ShapeDtypeStruct((n, 16), f32),
         jax.ShapeDtypeStruct((n, 1), f32)],
        data_x, w1cat, b1r, degacc)

    p1, m1 = _prop_kernel(n_pad, e_pad, 16)(rowpad(g1), src3, dst3, zz16)

    o_r, o_i, g2 = _tc_call(
        _t2_body,
        [jax.ShapeDtypeStruct((n, 16), f32),
         jax.ShapeDtypeStruct((n, 16), f32),
         jax.ShapeDtypeStruct((n, 32), f32)],
        y0b, p1[:, :n], m1[:, :n], dinv, w2cat, b2r)

    p2, m2 = _prop_kernel(n_pad, e_pad, 32)(rowpad(g2), src3, dst3, zz32)

    o_r3, o_i3, g3 = _tc_call(
        _t3_body,
        [jax.ShapeDtypeStruct((n, 8), f32),
         jax.ShapeDtypeStruct((n, 8), f32),
         jax.ShapeDtypeStruct((n, 16), f32)],
        o_r, o_i, p2[:, :n], m2[:, :n], dinv, w3cat, b3r)

    p3, m3 = _prop_kernel(n_pad, e_pad, 16)(rowpad(g3), src3, dst3, zz16)

    out = _tc_call(
        _t4_body,
        [jax.ShapeDtypeStruct((n, 1), f32)],
        o_r3, o_i3, p3[:, :n], m3[:, :n], dinv, Wr, brr)

    return out[0]


# speedup vs baseline: 47.2284x; 2.9370x over previous
import functools

import jax
import jax.numpy as jnp
from jax import lax
from jax.experimental import pallas as pl
from jax.experimental.pallas import tpu as pltpu
from jax.experimental.pallas import tpu_sc as plsc

NC = 2
NS = 16
NW = NC * NS
BLK = 128
CH = 8


def _pad_blocks(e):
    per_w = -(-e // (NW * BLK * CH)) * (BLK * CH)
    return per_w * NW


def _pad_nodes(n):
    return -(-n // (NS * 8)) * (NS * 8)


@functools.lru_cache(maxsize=None)
def _deg_kernel(n, e_pad):
    epw = e_pad // NW
    kpw = epw // BLK
    chd = 20
    rounds = kpw // chd
    rpt = n // NS
    n_acc = n + 16 * NW
    mesh = plsc.VectorSubcoreMesh(core_axis_name="c", subcore_axis_name="s",
                                  num_cores=NC, num_subcores=NS)

    def body(src3, dst3, zza, htab, ntab, out,
             acc, sall, dall, sibuf, halfbuf, negbuf, sem):
        c = lax.axis_index("c")
        s = lax.axis_index("s")
        wid = c * NS + s
        r0 = s * rpt
        apt = n_acc // NS
        pltpu.sync_copy(zza.at[pl.ds(s * apt, apt)],
                        acc.at[pl.ds(s * apt, apt)])
        pltpu.sync_copy(src3.at[pl.ds(wid * kpw, kpw)], sall)
        pltpu.sync_copy(dst3.at[pl.ds(wid * kpw, kpw)], dall)
        pltpu.sync_copy(htab, halfbuf)
        pltpu.sync_copy(ntab, negbuf)
        plsc.subcore_barrier()
        dummy16 = n + 16 * wid + lax.iota(jnp.int32, 16)

        for r in range(rounds):
            descs = []
            for j in range(chd):
                jj = r * chd + j
                for k in range(BLK // 16):
                    s16 = sall[jj, pl.ds(k * 16, 16)]
                    d16 = dall[jj, pl.ds(k * 16, 16)]
                    e16 = s16 == d16
                    sibuf[j, pl.ds(k * 16, 16)] = jnp.where(e16, s16,
                                                            dummy16)
                descs.append(pltpu.async_copy(halfbuf, acc.at[sall.at[jj]],
                                              sem, add=True))
                descs.append(pltpu.async_copy(halfbuf, acc.at[dall.at[jj]],
                                              sem, add=True))
                descs.append(pltpu.async_copy(negbuf, acc.at[sibuf.at[j]],
                                              sem, add=True))
            for d in descs:
                d.wait()

        plsc.subcore_barrier()
        pltpu.sync_copy(acc.at[pl.ds(r0, rpt)], out.at[c, pl.ds(r0, rpt)])

    return pl.kernel(
        body,
        out_type=jax.ShapeDtypeStruct((NC, n, 16), jnp.float32),
        mesh=mesh,
        compiler_params=pltpu.CompilerParams(use_tc_tiling_on_sc=False),
        scratch_types=[
            pltpu.VMEM_SHARED((n_acc, 16), jnp.float32),
            pltpu.VMEM((kpw, BLK), jnp.int32),
            pltpu.VMEM((kpw, BLK), jnp.int32),
            pltpu.VMEM((chd, BLK), jnp.int32),
            pltpu.VMEM((BLK, 16), jnp.float32),
            pltpu.VMEM((BLK, 16), jnp.float32),
            pltpu.SemaphoreType.DMA,
        ],
    )


@functools.lru_cache(maxsize=None)
def _prop_kernel(n, e_pad, f):
    epw = e_pad // NW
    kpw = epw // BLK
    ch = {16: 10, 32: 4}[f]
    pairs = kpw // (2 * ch)
    rpt = n // NS
    mesh = plsc.VectorSubcoreMesh(core_axis_name="c", subcore_axis_name="s",
                                  num_cores=NC, num_subcores=NS)

    def body(g, src3, dst3, zz, plus_o, minus_o,
             plus_s, minus_s, sall, dall, ra0, rb0, ra1, rb1, sem_g, sem_s):
        c = lax.axis_index("c")
        s = lax.axis_index("s")
        wid = c * NS + s
        r0 = s * rpt
        pltpu.sync_copy(zz.at[pl.ds(r0, rpt)], plus_s.at[pl.ds(r0, rpt)])
        pltpu.sync_copy(zz.at[pl.ds(r0, rpt)], minus_s.at[pl.ds(r0, rpt)])
        pltpu.sync_copy(src3.at[pl.ds(wid * kpw, kpw)], sall)
        pltpu.sync_copy(dst3.at[pl.ds(wid * kpw, kpw)], dall)
        plsc.subcore_barrier()

        def issue_g(base, ra, rb):
            ds_ = []
            for j in range(ch):
                jj = base + j
                ds_.append(pltpu.async_copy(g.at[dall.at[jj]], ra.at[j],
                                            sem_g))
                ds_.append(pltpu.async_copy(g.at[sall.at[jj]], rb.at[j],
                                            sem_g))
            return ds_

        def issue_s(base, ra, rb):
            ds_ = []
            for j in range(ch):
                jj = base + j
                ds_.append(pltpu.async_copy(ra.at[j],
                                            plus_s.at[sall.at[jj]],
                                            sem_s, add=True))
                ds_.append(pltpu.async_copy(rb.at[j],
                                            minus_s.at[dall.at[jj]],
                                            sem_s, add=True))
            return ds_

        def pair_body(t, _):
            b0 = t * (2 * ch)
            b1 = b0 + ch
            g0 = issue_g(b0, ra0, rb0)
            for d in g0:
                d.wait()
            g1 = issue_g(b1, ra1, rb1)
            s0 = issue_s(b0, ra0, rb0)
            for d in g1:
                d.wait()
            s1 = issue_s(b1, ra1, rb1)
            for d in s0:
                d.wait()
            for d in s1:
                d.wait()
            return 0

        lax.fori_loop(0, pairs, pair_body, 0)
        plsc.subcore_barrier()
        pltpu.sync_copy(plus_s.at[pl.ds(r0, rpt)],
                        plus_o.at[c, pl.ds(r0, rpt)])
        pltpu.sync_copy(minus_s.at[pl.ds(r0, rpt)],
                        minus_o.at[c, pl.ds(r0, rpt)])

    return pl.kernel(
        body,
        out_type=[jax.ShapeDtypeStruct((NC, n, f), jnp.float32),
                  jax.ShapeDtypeStruct((NC, n, f), jnp.float32)],
        mesh=mesh,
        compiler_params=pltpu.CompilerParams(use_tc_tiling_on_sc=False),
        scratch_types=[
            pltpu.VMEM_SHARED((n, f), jnp.float32),
            pltpu.VMEM_SHARED((n, f), jnp.float32),
            pltpu.VMEM((e_pad // NW // BLK, BLK), jnp.int32),
            pltpu.VMEM((e_pad // NW // BLK, BLK), jnp.int32),
            pltpu.VMEM((ch, BLK, f), jnp.float32),
            pltpu.VMEM((ch, BLK, f), jnp.float32),
            pltpu.VMEM((ch, BLK, f), jnp.float32),
            pltpu.VMEM((ch, BLK, f), jnp.float32),
            pltpu.SemaphoreType.DMA,
            pltpu.SemaphoreType.DMA,
        ],
    )


def _t1_body(x_ref, w_ref, b_ref, degacc_ref, y0b_ref, g1_ref, dinv_ref):
    y = jnp.dot(x_ref[...], w_ref[...], preferred_element_type=jnp.float32)
    dacc = degacc_ref[...]
    deg = dacc[0, :, 0] + dacc[1, :, 0]
    dinv = jnp.where(deg > 0, lax.rsqrt(deg), 0.0)[:, None]
    y0b_ref[...] = y[:, :16] + b_ref[...]
    g1_ref[...] = dinv * y[:, 16:32]
    dinv_ref[...] = dinv


def _t2_body(y0b_ref, p_ref, m_ref, dinv_ref, w_ref, b_ref,
             or_ref, oi_ref, g2_ref):
    dinv = dinv_ref[...]
    p = p_ref[...]
    m = m_ref[...]
    u = (0.5 * dinv) * (p[0] + p[1] - m[0] - m[1])
    y0b = y0b_ref[...]
    xr = jnp.maximum(y0b - u, 0.0)
    xi = y0b + u
    zr = jnp.dot(xr, w_ref[...], preferred_element_type=jnp.float32)
    zi = jnp.dot(xi, w_ref[...], preferred_element_type=jnp.float32)
    b = b_ref[...]
    or_ref[...] = zr[:, :16] + b
    oi_ref[...] = zi[:, :16] + b
    g2_ref[...] = dinv * jnp.concatenate([zr[:, 16:], zi[:, 16:]], axis=1)


def _t3_body(or_ref, oi_ref, p_ref, m_ref, dinv_ref, w_ref, b_ref,
             or3_ref, oi3_ref, g3_ref):
    dinv = dinv_ref[...]
    p = p_ref[...]
    m = m_ref[...]
    u = (0.5 * dinv) * (p[0] + p[1] - m[0] - m[1])
    xr = jnp.maximum(or_ref[...] - u[:, 16:32], 0.0)
    xi = oi_ref[...] + u[:, 0:16]
    zr = jnp.dot(xr, w_ref[...], preferred_element_type=jnp.float32)
    zi = jnp.dot(xi, w_ref[...], preferred_element_type=jnp.float32)
    b = b_ref[...]
    or3_ref[...] = zr[:, :8] + b
    oi3_ref[...] = zi[:, :8] + b
    g3_ref[...] = dinv * jnp.concatenate([zr[:, 8:], zi[:, 8:]], axis=1)


def _t4_body(or3_ref, oi3_ref, p_ref, m_ref, dinv_ref, wr_ref, br_ref,
             out_ref):
    dinv = dinv_ref[...]
    p = p_ref[...]
    m = m_ref[...]
    u = (0.5 * dinv) * (p[0] + p[1] - m[0] - m[1])
    xr = or3_ref[...] - u[:, 8:16]
    xi = oi3_ref[...] + u[:, 0:8]
    h = jnp.concatenate([xr, xi], axis=1)
    out_ref[...] = jnp.dot(h, wr_ref[...],
                           preferred_element_type=jnp.float32) + br_ref[...]


def _tc_call(body, out_shapes, *args):
    return pl.pallas_call(body, out_shape=out_shapes)(*args)


def kernel(data_x, data_edge_index, W1, b1, W2, b2, W3, b3, Wr, br):
    n, _ = data_x.shape
    e = data_edge_index.shape[1]
    e_pad = _pad_blocks(e)
    n_pad = _pad_nodes(n)

    src = data_edge_index[0]
    dst = data_edge_index[1]
    pad = e_pad - e
    zpad = jnp.zeros((pad,), jnp.int32)
    src3 = jnp.concatenate([src, zpad]).reshape(e_pad // BLK, BLK)
    dst3 = jnp.concatenate([dst, zpad]).reshape(e_pad // BLK, BLK)

    def rowpad(a):
        return jnp.concatenate(
            [a, jnp.zeros((n_pad - n, a.shape[1]), a.dtype)])

    zz16 = jnp.zeros((n_pad, 16), jnp.float32)
    zz32 = jnp.zeros((n_pad, 32), jnp.float32)
    zza = jnp.zeros((n_pad + 8 * NW, 16), jnp.float32)
    htab = jnp.zeros((BLK, 16), jnp.float32).at[:, 0].set(0.5)
    ntab = jnp.zeros((BLK, 16), jnp.float32).at[:, 0].set(-1.0)

    w1cat = jnp.concatenate([W1[0], W1[1]], axis=1)
    w2cat = jnp.concatenate([W2[0], W2[1]], axis=1)
    w3cat = jnp.concatenate([W3[0], W3[1]], axis=1)
    b1r = b1.reshape(1, -1)
    b2r = b2.reshape(1, -1)
    b3r = b3.reshape(1, -1)
    brr = br.reshape(1, -1)

    f32 = jnp.float32
    degacc = _deg_kernel(n_pad, e_pad)(src3, dst3, zza, htab, ntab)[:, :n]

    y0b, g1, dinv = _tc_call(
        _t1_body,
        [jax.ShapeDtypeStruct((n, 16), f32),
         jax.ShapeDtypeStruct((n, 16), f32),
         jax.ShapeDtypeStruct((n, 1), f32)],
        data_x, w1cat, b1r, degacc)

    p1, m1 = _prop_kernel(n_pad, e_pad, 16)(rowpad(g1), src3, dst3, zz16)

    o_r, o_i, g2 = _tc_call(
        _t2_body,
        [jax.ShapeDtypeStruct((n, 16), f32),
         jax.ShapeDtypeStruct((n, 16), f32),
         jax.ShapeDtypeStruct((n, 32), f32)],
        y0b, p1[:, :n], m1[:, :n], dinv, w2cat, b2r)

    p2, m2 = _prop_kernel(n_pad, e_pad, 32)(rowpad(g2), src3, dst3, zz32)

    o_r3, o_i3, g3 = _tc_call(
        _t3_body,
        [jax.ShapeDtypeStruct((n, 8), f32),
         jax.ShapeDtypeStruct((n, 8), f32),
         jax.ShapeDtypeStruct((n, 16), f32)],
        o_r, o_i, p2[:, :n], m2[:, :n], dinv, w3cat, b3r)

    p3, m3 = _prop_kernel(n_pad, e_pad, 16)(rowpad(g3), src3, dst3, zz16)

    out = _tc_call(
        _t4_body,
        [jax.ShapeDtypeStruct((n, 1), f32)],
        o_r3, o_i3, p3[:, :n], m3[:, :n], dinv, Wr, brr)

    return out[0]


# speedup vs baseline: 89.4780x; 1.8946x over previous
import functools

import jax
import jax.numpy as jnp
from jax import lax
from jax.experimental import pallas as pl
from jax.experimental.pallas import tpu as pltpu
from jax.experimental.pallas import tpu_sc as plsc

NC = 2
NS = 16
NW = NC * NS
BLK = 128
CH = 8


def _pad_blocks(e):
    per_w = -(-e // (NW * BLK * CH)) * (BLK * CH)
    return per_w * NW


def _pad_nodes(n):
    return -(-n // (NS * 8)) * (NS * 8)


@functools.lru_cache(maxsize=None)
def _deg_kernel(n, e_pad):
    epw = e_pad // NW
    kpw = epw // BLK
    chd = 20
    rounds = kpw // chd
    rpt = n // NS
    n_acc = n + 16 * NW
    mesh = plsc.VectorSubcoreMesh(core_axis_name="c", subcore_axis_name="s",
                                  num_cores=NC, num_subcores=NS)

    def body(src3, dst3, zza, htab, ntab, out,
             acc, sall, dall, sibuf, halfbuf, negbuf, sem):
        c = lax.axis_index("c")
        s = lax.axis_index("s")
        wid = c * NS + s
        r0 = s * rpt
        apt = n_acc // NS
        pltpu.sync_copy(zza.at[pl.ds(s * apt, apt)],
                        acc.at[pl.ds(s * apt, apt)])
        pltpu.sync_copy(src3.at[pl.ds(wid * kpw, kpw)], sall)
        pltpu.sync_copy(dst3.at[pl.ds(wid * kpw, kpw)], dall)
        pltpu.sync_copy(htab, halfbuf)
        pltpu.sync_copy(ntab, negbuf)
        plsc.subcore_barrier()
        dummy16 = n + 16 * wid + lax.iota(jnp.int32, 16)

        for r in range(rounds):
            descs = []
            for j in range(chd):
                jj = r * chd + j
                for k in range(BLK // 16):
                    s16 = sall[jj, pl.ds(k * 16, 16)]
                    d16 = dall[jj, pl.ds(k * 16, 16)]
                    e16 = s16 == d16
                    sibuf[j, pl.ds(k * 16, 16)] = jnp.where(e16, s16,
                                                            dummy16)
                descs.append(pltpu.async_copy(halfbuf, acc.at[sall.at[jj]],
                                              sem, add=True))
                descs.append(pltpu.async_copy(halfbuf, acc.at[dall.at[jj]],
                                              sem, add=True))
                descs.append(pltpu.async_copy(negbuf, acc.at[sibuf.at[j]],
                                              sem, add=True))
            for d in descs:
                d.wait()

        plsc.subcore_barrier()
        pltpu.sync_copy(acc.at[pl.ds(r0, rpt)], out.at[c, pl.ds(r0, rpt)])

    return pl.kernel(
        body,
        out_type=jax.ShapeDtypeStruct((NC, n, 16), jnp.float32),
        mesh=mesh,
        compiler_params=pltpu.CompilerParams(use_tc_tiling_on_sc=False),
        scratch_types=[
            pltpu.VMEM_SHARED((n_acc, 16), jnp.float32),
            pltpu.VMEM((kpw, BLK), jnp.int32),
            pltpu.VMEM((kpw, BLK), jnp.int32),
            pltpu.VMEM((chd, BLK), jnp.int32),
            pltpu.VMEM((BLK, 16), jnp.float32),
            pltpu.VMEM((BLK, 16), jnp.float32),
            pltpu.SemaphoreType.DMA,
        ],
    )


@functools.lru_cache(maxsize=None)
def _prop_kernel(n, e_pad, f):
    epw = e_pad // NW
    kpw = epw // BLK
    ch = {16: 10, 32: 4}[f]
    pairs = kpw // (2 * ch)
    rpt = n // NS
    mesh = plsc.VectorSubcoreMesh(core_axis_name="c", subcore_axis_name="s",
                                  num_cores=NC, num_subcores=NS)

    def body(g, src3, dst3, zz, plus_o, minus_o,
             plus_s, minus_s, sall, dall, ra0, rb0, ra1, rb1, sem_g, sem_s):
        c = lax.axis_index("c")
        s = lax.axis_index("s")
        wid = c * NS + s
        r0 = s * rpt
        pltpu.sync_copy(zz.at[pl.ds(r0, rpt)], plus_s.at[pl.ds(r0, rpt)])
        pltpu.sync_copy(zz.at[pl.ds(r0, rpt)], minus_s.at[pl.ds(r0, rpt)])
        pltpu.sync_copy(src3.at[pl.ds(wid * kpw, kpw)], sall)
        pltpu.sync_copy(dst3.at[pl.ds(wid * kpw, kpw)], dall)
        plsc.subcore_barrier()

        def issue_g(base, ra, rb):
            ds_ = []
            for j in range(ch):
                jj = base + j
                ds_.append(pltpu.async_copy(g.at[dall.at[jj]], ra.at[j],
                                            sem_g))
                ds_.append(pltpu.async_copy(g.at[sall.at[jj]], rb.at[j],
                                            sem_g))
            return ds_

        def issue_s(base, ra, rb):
            ds_ = []
            for j in range(ch):
                jj = base + j
                ds_.append(pltpu.async_copy(ra.at[j],
                                            plus_s.at[sall.at[jj]],
                                            sem_s, add=True))
                ds_.append(pltpu.async_copy(rb.at[j],
                                            minus_s.at[dall.at[jj]],
                                            sem_s, add=True))
            return ds_

        def pair_body(t, _):
            b0 = t * (2 * ch)
            b1 = b0 + ch
            g0 = issue_g(b0, ra0, rb0)
            for d in g0:
                d.wait()
            g1 = issue_g(b1, ra1, rb1)
            s0 = issue_s(b0, ra0, rb0)
            for d in g1:
                d.wait()
            s1 = issue_s(b1, ra1, rb1)
            for d in s0:
                d.wait()
            for d in s1:
                d.wait()
            return 0

        lax.fori_loop(0, pairs, pair_body, 0)
        plsc.subcore_barrier()
        pltpu.sync_copy(plus_s.at[pl.ds(r0, rpt)],
                        plus_o.at[c, pl.ds(r0, rpt)])
        pltpu.sync_copy(minus_s.at[pl.ds(r0, rpt)],
                        minus_o.at[c, pl.ds(r0, rpt)])

    return pl.kernel(
        body,
        out_type=[jax.ShapeDtypeStruct((NC, n, f), jnp.float32),
                  jax.ShapeDtypeStruct((NC, n, f), jnp.float32)],
        mesh=mesh,
        compiler_params=pltpu.CompilerParams(use_tc_tiling_on_sc=False),
        scratch_types=[
            pltpu.VMEM_SHARED((n, f), jnp.float32),
            pltpu.VMEM_SHARED((n, f), jnp.float32),
            pltpu.VMEM((e_pad // NW // BLK, BLK), jnp.int32),
            pltpu.VMEM((e_pad // NW // BLK, BLK), jnp.int32),
            pltpu.VMEM((ch, BLK, f), jnp.float32),
            pltpu.VMEM((ch, BLK, f), jnp.float32),
            pltpu.VMEM((ch, BLK, f), jnp.float32),
            pltpu.VMEM((ch, BLK, f), jnp.float32),
            pltpu.SemaphoreType.DMA,
            pltpu.SemaphoreType.DMA,
        ],
    )


def _t1_body(x_ref, w_ref, b_ref, degacc_ref, y0b_ref, g1_ref, dinv_ref):
    y = jnp.dot(x_ref[...], w_ref[...], preferred_element_type=jnp.float32)
    dacc = degacc_ref[...]
    deg = dacc[0, :, 0] + dacc[1, :, 0]
    dinv = jnp.where(deg > 0, lax.rsqrt(deg), 0.0)[:, None]
    y0b_ref[...] = y[:, :16] + b_ref[...]
    g1_ref[...] = dinv * y[:, 16:32]
    dinv_ref[...] = dinv


def _t2_body(y0b_ref, p_ref, m_ref, dinv_ref, w_ref, b_ref,
             or_ref, oi_ref, g2_ref):
    dinv = dinv_ref[...]
    p = p_ref[...]
    m = m_ref[...]
    u = (0.5 * dinv) * (p[0] + p[1] - m[0] - m[1])
    y0b = y0b_ref[...]
    xr = jnp.maximum(y0b - u, 0.0)
    xi = y0b + u
    zr = jnp.dot(xr, w_ref[...], preferred_element_type=jnp.float32)
    zi = jnp.dot(xi, w_ref[...], preferred_element_type=jnp.float32)
    b = b_ref[...]
    or_ref[...] = zr[:, :16] + b
    oi_ref[...] = zi[:, :16] + b
    g2_ref[...] = dinv * jnp.concatenate([zr[:, 16:], zi[:, 16:]], axis=1)


def _t3_body(or_ref, oi_ref, p_ref, m_ref, dinv_ref, w_ref, b_ref,
             or3_ref, oi3_ref, g3_ref):
    dinv = dinv_ref[...]
    p = p_ref[...]
    m = m_ref[...]
    u = (0.5 * dinv) * (p[0] + p[1] - m[0] - m[1])
    xr = jnp.maximum(or_ref[...] - u[:, 16:32], 0.0)
    xi = oi_ref[...] + u[:, 0:16]
    zr = jnp.dot(xr, w_ref[...], preferred_element_type=jnp.float32)
    zi = jnp.dot(xi, w_ref[...], preferred_element_type=jnp.float32)
    b = b_ref[...]
    or3_ref[...] = zr[:, :8] + b
    oi3_ref[...] = zi[:, :8] + b
    g3_ref[...] = dinv * jnp.concatenate([zr[:, 8:], zi[:, 8:]], axis=1)


def _t4_body(or3_ref, oi3_ref, p_ref, m_ref, dinv_ref, wr_ref, br_ref,
             out_ref):
    dinv = dinv_ref[...]
    p = p_ref[...]
    m = m_ref[...]
    u = (0.5 * dinv) * (p[0] + p[1] - m[0] - m[1])
    xr = or3_ref[...] - u[:, 8:16]
    xi = oi3_ref[...] + u[:, 0:8]
    h = jnp.concatenate([xr, xi], axis=1)
    out_ref[...] = jnp.dot(h, wr_ref[...],
                           preferred_element_type=jnp.float32) + br_ref[...]


def _tc_call(body, out_shapes, *args):
    return pl.pallas_call(body, out_shape=out_shapes)(*args)


def kernel(data_x, data_edge_index, W1, b1, W2, b2, W3, b3, Wr, br):
    n, _ = data_x.shape
    e = data_edge_index.shape[1]
    e_pad = _pad_blocks(e)
    n_pad = _pad_nodes(n)

    src = data_edge_index[0]
    dst = data_edge_index[1]
    pad = e_pad - e
    zpad = n + lax.iota(jnp.int32, pad) % (n_pad - n)
    src3 = jnp.concatenate([src, zpad]).reshape(e_pad // BLK, BLK)
    dst3 = jnp.concatenate([dst, zpad]).reshape(e_pad // BLK, BLK)

    def rowpad(a):
        return jnp.concatenate(
            [a, jnp.zeros((n_pad - n, a.shape[1]), a.dtype)])

    zz16 = jnp.zeros((n_pad, 16), jnp.float32)
    zz32 = jnp.zeros((n_pad, 32), jnp.float32)
    zza = jnp.zeros((n_pad + 8 * NW, 16), jnp.float32)
    htab = jnp.zeros((BLK, 16), jnp.float32).at[:, 0].set(0.5)
    ntab = jnp.zeros((BLK, 16), jnp.float32).at[:, 0].set(-1.0)

    w1cat = jnp.concatenate([W1[0], W1[1]], axis=1)
    w2cat = jnp.concatenate([W2[0], W2[1]], axis=1)
    w3cat = jnp.concatenate([W3[0], W3[1]], axis=1)
    b1r = b1.reshape(1, -1)
    b2r = b2.reshape(1, -1)
    b3r = b3.reshape(1, -1)
    brr = br.reshape(1, -1)

    f32 = jnp.float32
    degacc = _deg_kernel(n_pad, e_pad)(src3, dst3, zza, htab, ntab)[:, :n]

    y0b, g1, dinv = _tc_call(
        _t1_body,
        [jax.ShapeDtypeStruct((n, 16), f32),
         jax.ShapeDtypeStruct((n, 16), f32),
         jax.ShapeDtypeStruct((n, 1), f32)],
        data_x, w1cat, b1r, degacc)

    p1, m1 = _prop_kernel(n_pad, e_pad, 16)(rowpad(g1), src3, dst3, zz16)

    o_r, o_i, g2 = _tc_call(
        _t2_body,
        [jax.ShapeDtypeStruct((n, 16), f32),
         jax.ShapeDtypeStruct((n, 16), f32),
         jax.ShapeDtypeStruct((n, 32), f32)],
        y0b, p1[:, :n], m1[:, :n], dinv, w2cat, b2r)

    p2, m2 = _prop_kernel(n_pad, e_pad, 32)(rowpad(g2), src3, dst3, zz32)

    o_r3, o_i3, g3 = _tc_call(
        _t3_body,
        [jax.ShapeDtypeStruct((n, 8), f32),
         jax.ShapeDtypeStruct((n, 8), f32),
         jax.ShapeDtypeStruct((n, 16), f32)],
        o_r, o_i, p2[:, :n], m2[:, :n], dinv, w3cat, b3r)

    p3, m3 = _prop_kernel(n_pad, e_pad, 16)(rowpad(g3), src3, dst3, zz16)

    out = _tc_call(
        _t4_body,
        [jax.ShapeDtypeStruct((n, 1), f32)],
        o_r3, o_i3, p3[:, :n], m3[:, :n], dinv, Wr, brr)

    return out[0]


# speedup vs baseline: 101.6034x; 1.1355x over previous
import functools

import jax
import jax.numpy as jnp
from jax import lax
from jax.experimental import pallas as pl
from jax.experimental.pallas import tpu as pltpu
from jax.experimental.pallas import tpu_sc as plsc

NC = 2
NS = 16
NW = NC * NS
BLK = 128
CH = 8


def _pad_blocks(e):
    per_w = -(-e // (NW * BLK * CH)) * (BLK * CH)
    return per_w * NW


def _pad_nodes(n):
    return -(-n // (NS * 8)) * (NS * 8)


@functools.lru_cache(maxsize=None)
def _deg_kernel(n, e_pad):
    epw = e_pad // NW
    kpw = epw // BLK
    chd = 20
    rounds = kpw // chd
    rpt = n // NS
    n_acc = n + 16 * NW
    mesh = plsc.VectorSubcoreMesh(core_axis_name="c", subcore_axis_name="s",
                                  num_cores=NC, num_subcores=NS)

    def body(src3, dst3, zza, htab, ntab, out,
             acc, sall, dall, sibuf, halfbuf, negbuf, sem):
        c = lax.axis_index("c")
        s = lax.axis_index("s")
        wid = c * NS + s
        r0 = s * rpt
        apt = n_acc // NS
        pltpu.sync_copy(zza.at[pl.ds(s * apt, apt)],
                        acc.at[pl.ds(s * apt, apt)])
        pltpu.sync_copy(src3.at[pl.ds(wid * kpw, kpw)], sall)
        pltpu.sync_copy(dst3.at[pl.ds(wid * kpw, kpw)], dall)
        pltpu.sync_copy(htab, halfbuf)
        pltpu.sync_copy(ntab, negbuf)
        plsc.subcore_barrier()
        dummy16 = n + 16 * wid + lax.iota(jnp.int32, 16)

        for r in range(rounds):
            descs = []
            for j in range(chd):
                jj = r * chd + j
                for k in range(BLK // 16):
                    s16 = sall[jj, pl.ds(k * 16, 16)]
                    d16 = dall[jj, pl.ds(k * 16, 16)]
                    e16 = s16 == d16
                    sibuf[j, pl.ds(k * 16, 16)] = jnp.where(e16, s16,
                                                            dummy16)
                descs.append(pltpu.async_copy(halfbuf, acc.at[sall.at[jj]],
                                              sem, add=True))
                descs.append(pltpu.async_copy(halfbuf, acc.at[dall.at[jj]],
                                              sem, add=True))
                descs.append(pltpu.async_copy(negbuf, acc.at[sibuf.at[j]],
                                              sem, add=True))
            for d in descs:
                d.wait()

        plsc.subcore_barrier()
        pltpu.sync_copy(acc.at[pl.ds(r0, rpt)], out.at[c, pl.ds(r0, rpt)])

    return pl.kernel(
        body,
        out_type=jax.ShapeDtypeStruct((NC, n, 16), jnp.float32),
        mesh=mesh,
        compiler_params=pltpu.CompilerParams(use_tc_tiling_on_sc=False),
        scratch_types=[
            pltpu.VMEM_SHARED((n_acc, 16), jnp.float32),
            pltpu.VMEM((kpw, BLK), jnp.int32),
            pltpu.VMEM((kpw, BLK), jnp.int32),
            pltpu.VMEM((chd, BLK), jnp.int32),
            pltpu.VMEM((BLK, 16), jnp.float32),
            pltpu.VMEM((BLK, 16), jnp.float32),
            pltpu.SemaphoreType.DMA,
        ],
    )


@functools.lru_cache(maxsize=None)
def _prop_kernel(n, e_pad, f):
    epw = e_pad // NW
    kpw = epw // BLK
    ch = {16: 10, 32: 4}[f]
    pairs = kpw // (2 * ch)
    rpt = n // NS
    mesh = plsc.VectorSubcoreMesh(core_axis_name="c", subcore_axis_name="s",
                                  num_cores=NC, num_subcores=NS)

    def body(g, src3, dst3, zz, plus_o, minus_o,
             plus_s, minus_s, sall, dall, ra0, rb0, ra1, rb1, sem_g, sem_s):
        c = lax.axis_index("c")
        s = lax.axis_index("s")
        wid = c * NS + s
        r0 = s * rpt
        pltpu.sync_copy(zz.at[pl.ds(r0, rpt)], plus_s.at[pl.ds(r0, rpt)])
        pltpu.sync_copy(zz.at[pl.ds(r0, rpt)], minus_s.at[pl.ds(r0, rpt)])
        pltpu.sync_copy(src3.at[pl.ds(wid * kpw, kpw)], sall)
        pltpu.sync_copy(dst3.at[pl.ds(wid * kpw, kpw)], dall)
        plsc.subcore_barrier()

        def issue_g(base, ra, rb):
            ds_ = []
            for j in range(ch):
                jj = base + j
                ds_.append(pltpu.async_copy(g.at[dall.at[jj]], ra.at[j],
                                            sem_g))
                ds_.append(pltpu.async_copy(g.at[sall.at[jj]], rb.at[j],
                                            sem_g))
            return ds_

        def issue_s(base, ra, rb):
            ds_ = []
            for j in range(ch):
                jj = base + j
                ds_.append(pltpu.async_copy(ra.at[j],
                                            plus_s.at[sall.at[jj]],
                                            sem_s, add=True))
                ds_.append(pltpu.async_copy(rb.at[j],
                                            minus_s.at[dall.at[jj]],
                                            sem_s, add=True))
            return ds_

        def pair_body(t, _):
            b0 = t * (2 * ch)
            b1 = b0 + ch
            g0 = issue_g(b0, ra0, rb0)
            for d in g0:
                d.wait()
            g1 = issue_g(b1, ra1, rb1)
            s0 = issue_s(b0, ra0, rb0)
            for d in g1:
                d.wait()
            s1 = issue_s(b1, ra1, rb1)
            for d in s0:
                d.wait()
            for d in s1:
                d.wait()
            return 0

        lax.fori_loop(0, pairs, pair_body, 0)
        plsc.subcore_barrier()
        pltpu.sync_copy(plus_s.at[pl.ds(r0, rpt)],
                        plus_o.at[c, pl.ds(r0, rpt)])
        pltpu.sync_copy(minus_s.at[pl.ds(r0, rpt)],
                        minus_o.at[c, pl.ds(r0, rpt)])

    return pl.kernel(
        body,
        out_type=[jax.ShapeDtypeStruct((NC, n, f), jnp.float32),
                  jax.ShapeDtypeStruct((NC, n, f), jnp.float32)],
        mesh=mesh,
        compiler_params=pltpu.CompilerParams(use_tc_tiling_on_sc=False),
        scratch_types=[
            pltpu.VMEM_SHARED((n, f), jnp.float32),
            pltpu.VMEM_SHARED((n, f), jnp.float32),
            pltpu.VMEM((e_pad // NW // BLK, BLK), jnp.int32),
            pltpu.VMEM((e_pad // NW // BLK, BLK), jnp.int32),
            pltpu.VMEM((ch, BLK, f), jnp.float32),
            pltpu.VMEM((ch, BLK, f), jnp.float32),
            pltpu.VMEM((ch, BLK, f), jnp.float32),
            pltpu.VMEM((ch, BLK, f), jnp.float32),
            pltpu.SemaphoreType.DMA,
            pltpu.SemaphoreType.DMA,
        ],
    )


def _t1_body(x_ref, w_ref, b_ref, degacc_ref, y0b_ref, g1_ref, dinv_ref):
    n_pad = degacc_ref.shape[1]
    n = x_ref.shape[0]
    y = jnp.dot(x_ref[...], w_ref[...], preferred_element_type=jnp.float32)
    y = jnp.concatenate(
        [y, jnp.zeros((n_pad - n, y.shape[1]), jnp.float32)])
    dacc = degacc_ref[...]
    deg = dacc[0, :, 0] + dacc[1, :, 0]
    dinv = jnp.where(deg > 0, lax.rsqrt(deg), 0.0)[:, None]
    y0b_ref[...] = y[:, :16] + b_ref[...]
    g1_ref[...] = dinv * y[:, 16:32]
    dinv_ref[...] = dinv


def _t2_body(y0b_ref, p_ref, m_ref, dinv_ref, w_ref, b_ref,
             or_ref, oi_ref, g2_ref):
    dinv = dinv_ref[...]
    p = p_ref[...]
    m = m_ref[...]
    u = (0.5 * dinv) * (p[0] + p[1] - m[0] - m[1])
    y0b = y0b_ref[...]
    xr = jnp.maximum(y0b - u, 0.0)
    xi = y0b + u
    zr = jnp.dot(xr, w_ref[...], preferred_element_type=jnp.float32)
    zi = jnp.dot(xi, w_ref[...], preferred_element_type=jnp.float32)
    b = b_ref[...]
    or_ref[...] = zr[:, :16] + b
    oi_ref[...] = zi[:, :16] + b
    g2_ref[...] = dinv * jnp.concatenate([zr[:, 16:], zi[:, 16:]], axis=1)


def _t3_body(or_ref, oi_ref, p_ref, m_ref, dinv_ref, w_ref, b_ref,
             or3_ref, oi3_ref, g3_ref):
    dinv = dinv_ref[...]
    p = p_ref[...]
    m = m_ref[...]
    u = (0.5 * dinv) * (p[0] + p[1] - m[0] - m[1])
    xr = jnp.maximum(or_ref[...] - u[:, 16:32], 0.0)
    xi = oi_ref[...] + u[:, 0:16]
    zr = jnp.dot(xr, w_ref[...], preferred_element_type=jnp.float32)
    zi = jnp.dot(xi, w_ref[...], preferred_element_type=jnp.float32)
    b = b_ref[...]
    or3_ref[...] = zr[:, :8] + b
    oi3_ref[...] = zi[:, :8] + b
    g3_ref[...] = dinv * jnp.concatenate([zr[:, 8:], zi[:, 8:]], axis=1)


def _t4_body(or3_ref, oi3_ref, p_ref, m_ref, dinv_ref, wr_ref, br_ref,
             out_ref):
    dinv = dinv_ref[...]
    p = p_ref[...]
    m = m_ref[...]
    u = (0.5 * dinv) * (p[0] + p[1] - m[0] - m[1])
    xr = or3_ref[...] - u[:, 8:16]
    xi = oi3_ref[...] + u[:, 0:8]
    h = jnp.concatenate([xr, xi], axis=1)
    res = jnp.dot(h, wr_ref[...],
                  preferred_element_type=jnp.float32) + br_ref[...]
    out_ref[...] = res[:out_ref.shape[0]]


def _tc_call(body, out_shapes, *args):
    return pl.pallas_call(
        body, out_shape=out_shapes,
        compiler_params=pltpu.CompilerParams(
            vmem_limit_bytes=100 * 1024 * 1024))(*args)


def kernel(data_x, data_edge_index, W1, b1, W2, b2, W3, b3, Wr, br):
    n, _ = data_x.shape
    e = data_edge_index.shape[1]
    e_pad = _pad_blocks(e)
    n_pad = _pad_nodes(n)

    src = data_edge_index[0]
    dst = data_edge_index[1]
    pad = e_pad - e
    zpad = n + lax.iota(jnp.int32, pad) % (n_pad - n)
    src3 = jnp.concatenate([src, zpad]).reshape(e_pad // BLK, BLK)
    dst3 = jnp.concatenate([dst, zpad]).reshape(e_pad // BLK, BLK)

    zz16 = jnp.zeros((n_pad, 16), jnp.float32)
    zz32 = jnp.zeros((n_pad, 32), jnp.float32)
    zza = jnp.zeros((n_pad + 8 * NW, 16), jnp.float32)
    htab = jnp.zeros((BLK, 16), jnp.float32).at[:, 0].set(0.5)
    ntab = jnp.zeros((BLK, 16), jnp.float32).at[:, 0].set(-1.0)

    w1cat = jnp.concatenate([W1[0], W1[1]], axis=1)
    w2cat = jnp.concatenate([W2[0], W2[1]], axis=1)
    w3cat = jnp.concatenate([W3[0], W3[1]], axis=1)
    b1r = b1.reshape(1, -1)
    b2r = b2.reshape(1, -1)
    b3r = b3.reshape(1, -1)
    brr = br.reshape(1, -1)

    f32 = jnp.float32
    degacc = _deg_kernel(n_pad, e_pad)(src3, dst3, zza, htab, ntab)

    y0b, g1, dinv = _tc_call(
        _t1_body,
        [jax.ShapeDtypeStruct((n_pad, 16), f32),
         jax.ShapeDtypeStruct((n_pad, 16), f32),
         jax.ShapeDtypeStruct((n_pad, 1), f32)],
        data_x, w1cat, b1r, degacc)

    p1, m1 = _prop_kernel(n_pad, e_pad, 16)(g1, src3, dst3, zz16)

    o_r, o_i, g2 = _tc_call(
        _t2_body,
        [jax.ShapeDtypeStruct((n_pad, 16), f32),
         jax.ShapeDtypeStruct((n_pad, 16), f32),
         jax.ShapeDtypeStruct((n_pad, 32), f32)],
        y0b, p1, m1, dinv, w2cat, b2r)

    p2, m2 = _prop_kernel(n_pad, e_pad, 32)(g2, src3, dst3, zz32)

    o_r3, o_i3, g3 = _tc_call(
        _t3_body,
        [jax.ShapeDtypeStruct((n_pad, 8), f32),
         jax.ShapeDtypeStruct((n_pad, 8), f32),
         jax.ShapeDtypeStruct((n_pad, 16), f32)],
        o_r, o_i, p2, m2, dinv, w3cat, b3r)

    p3, m3 = _prop_kernel(n_pad, e_pad, 16)(g3, src3, dst3, zz16)

    out = _tc_call(
        _t4_body,
        [jax.ShapeDtypeStruct((n, 1), f32)],
        o_r3, o_i3, p3, m3, dinv, Wr, brr)

    return out[0]


# speedup vs baseline: 130.7401x; 1.2868x over previous
import functools

import jax
import jax.numpy as jnp
from jax import lax
from jax.experimental import pallas as pl
from jax.experimental.pallas import tpu as pltpu
from jax.experimental.pallas import tpu_sc as plsc

NC = 2
NS = 16
NW = NC * NS
BLK = 128
CH = 8


def _pad_blocks(e):
    per_w = -(-e // (NW * BLK * CH)) * (BLK * CH)
    return per_w * NW


def _pad_nodes(n):
    return -(-n // (NS * 8)) * (NS * 8)


@functools.lru_cache(maxsize=None)
def _deg_kernel(n, e_pad):
    epw = e_pad // NW
    kpw = epw // BLK
    chd = 20
    rounds = kpw // chd
    rpt = n // NS
    n_acc = n + 16 * NW
    mesh = plsc.VectorSubcoreMesh(core_axis_name="c", subcore_axis_name="s",
                                  num_cores=NC, num_subcores=NS)

    def body(src3, dst3, zza, htab, ntab, out,
             acc, sall, dall, sibuf, halfbuf, negbuf, sem):
        c = lax.axis_index("c")
        s = lax.axis_index("s")
        wid = c * NS + s
        r0 = s * rpt
        apt = n_acc // NS
        pltpu.sync_copy(zza.at[pl.ds(s * apt, apt)],
                        acc.at[pl.ds(s * apt, apt)])
        pltpu.sync_copy(src3.at[pl.ds(wid * kpw, kpw)], sall)
        pltpu.sync_copy(dst3.at[pl.ds(wid * kpw, kpw)], dall)
        pltpu.sync_copy(htab, halfbuf)
        pltpu.sync_copy(ntab, negbuf)
        plsc.subcore_barrier()
        dummy16 = n + 16 * wid + lax.iota(jnp.int32, 16)

        for r in range(rounds):
            descs = []
            for j in range(chd):
                jj = r * chd + j
                for k in range(BLK // 16):
                    s16 = sall[jj, pl.ds(k * 16, 16)]
                    d16 = dall[jj, pl.ds(k * 16, 16)]
                    e16 = s16 == d16
                    sibuf[j, pl.ds(k * 16, 16)] = jnp.where(e16, s16,
                                                            dummy16)
                descs.append(pltpu.async_copy(halfbuf, acc.at[sall.at[jj]],
                                              sem, add=True))
                descs.append(pltpu.async_copy(halfbuf, acc.at[dall.at[jj]],
                                              sem, add=True))
                descs.append(pltpu.async_copy(negbuf, acc.at[sibuf.at[j]],
                                              sem, add=True))
            for d in descs:
                d.wait()

        plsc.subcore_barrier()
        pltpu.sync_copy(acc.at[pl.ds(r0, rpt)], out.at[c, pl.ds(r0, rpt)])

    return pl.kernel(
        body,
        out_type=jax.ShapeDtypeStruct((NC, n, 16), jnp.float32),
        mesh=mesh,
        compiler_params=pltpu.CompilerParams(use_tc_tiling_on_sc=False),
        scratch_types=[
            pltpu.VMEM_SHARED((n_acc, 16), jnp.float32),
            pltpu.VMEM((kpw, BLK), jnp.int32),
            pltpu.VMEM((kpw, BLK), jnp.int32),
            pltpu.VMEM((chd, BLK), jnp.int32),
            pltpu.VMEM((BLK, 16), jnp.float32),
            pltpu.VMEM((BLK, 16), jnp.float32),
            pltpu.SemaphoreType.DMA,
        ],
    )


@functools.lru_cache(maxsize=None)
def _prop_kernel(n, e_pad, f):
    epw = e_pad // NW
    kpw = epw // BLK
    ch = {16: 10, 32: 4}[f]
    pairs = kpw // (2 * ch)
    rpt = n // NS
    mesh = plsc.VectorSubcoreMesh(core_axis_name="c", subcore_axis_name="s",
                                  num_cores=NC, num_subcores=NS)

    def body(g, src3, dst3, zz, plus_o, minus_o,
             plus_s, minus_s, sall, dall, ra0, rb0, ra1, rb1, sem_g, sem_s):
        c = lax.axis_index("c")
        s = lax.axis_index("s")
        wid = c * NS + s
        r0 = s * rpt
        pltpu.sync_copy(zz.at[pl.ds(r0, rpt)], plus_s.at[pl.ds(r0, rpt)])
        pltpu.sync_copy(zz.at[pl.ds(r0, rpt)], minus_s.at[pl.ds(r0, rpt)])
        pltpu.sync_copy(src3.at[pl.ds(wid * kpw, kpw)], sall)
        pltpu.sync_copy(dst3.at[pl.ds(wid * kpw, kpw)], dall)
        plsc.subcore_barrier()

        def issue_g(base, ra, rb):
            ds_ = []
            for j in range(ch):
                jj = base + j
                ds_.append(pltpu.async_copy(g.at[dall.at[jj]], ra.at[j],
                                            sem_g))
                ds_.append(pltpu.async_copy(g.at[sall.at[jj]], rb.at[j],
                                            sem_g))
            return ds_

        def issue_s(base, ra, rb):
            ds_ = []
            for j in range(ch):
                jj = base + j
                ds_.append(pltpu.async_copy(ra.at[j],
                                            plus_s.at[sall.at[jj]],
                                            sem_s, add=True))
                ds_.append(pltpu.async_copy(rb.at[j],
                                            minus_s.at[dall.at[jj]],
                                            sem_s, add=True))
            return ds_

        def pair_body(t, _):
            b0 = t * (2 * ch)
            b1 = b0 + ch
            g0 = issue_g(b0, ra0, rb0)
            for d in g0:
                d.wait()
            g1 = issue_g(b1, ra1, rb1)
            s0 = issue_s(b0, ra0, rb0)
            for d in g1:
                d.wait()
            s1 = issue_s(b1, ra1, rb1)
            for d in s0:
                d.wait()
            for d in s1:
                d.wait()
            return 0

        lax.fori_loop(0, pairs, pair_body, 0)
        plsc.subcore_barrier()
        pltpu.sync_copy(plus_s.at[pl.ds(r0, rpt)],
                        plus_o.at[c, pl.ds(r0, rpt)])
        pltpu.sync_copy(minus_s.at[pl.ds(r0, rpt)],
                        minus_o.at[c, pl.ds(r0, rpt)])

    return pl.kernel(
        body,
        out_type=[jax.ShapeDtypeStruct((NC, n, f), jnp.float32),
                  jax.ShapeDtypeStruct((NC, n, f), jnp.float32)],
        mesh=mesh,
        compiler_params=pltpu.CompilerParams(use_tc_tiling_on_sc=False),
        scratch_types=[
            pltpu.VMEM_SHARED((n, f), jnp.float32),
            pltpu.VMEM_SHARED((n, f), jnp.float32),
            pltpu.VMEM((e_pad // NW // BLK, BLK), jnp.int32),
            pltpu.VMEM((e_pad // NW // BLK, BLK), jnp.int32),
            pltpu.VMEM((ch, BLK, f), jnp.float32),
            pltpu.VMEM((ch, BLK, f), jnp.float32),
            pltpu.VMEM((ch, BLK, f), jnp.float32),
            pltpu.VMEM((ch, BLK, f), jnp.float32),
            pltpu.SemaphoreType.DMA,
            pltpu.SemaphoreType.DMA,
        ],
    )


def _t1_body(x_ref, wb0_ref, wb1_ref, s16_ref, b_ref, degacc_ref,
             y0b_ref, g1_ref, dinv_ref):
    nv = degacc_ref.shape[1]
    n_pad = nv * 8
    n = x_ref.shape[0]
    x = x_ref[...]
    x = jnp.concatenate([x, jnp.zeros((n_pad - n, 128), jnp.float32)])
    xg = x.reshape(nv, 8, 128).reshape(nv, 1024)
    y0bv = jnp.dot(xg, wb0_ref[...], preferred_element_type=jnp.float32,
                   precision=lax.Precision.HIGHEST)
    y1v = jnp.dot(xg, wb1_ref[...], preferred_element_type=jnp.float32,
                   precision=lax.Precision.HIGHEST)
    dacc = degacc_ref[...]
    dv = dacc[0] + dacc[1]
    degb = jnp.dot(dv, s16_ref[...], preferred_element_type=jnp.float32,
                   precision=lax.Precision.HIGHEST)
    dinv16 = jnp.where(degb > 0, lax.rsqrt(degb), 0.0)
    y0b_ref[...] = y0bv + b_ref[...]
    g1_ref[...] = dinv16 * y1v
    dinv_ref[...] = dinv16


def _t2_body(y0b_ref, p_ref, m_ref, dinv_ref, k2_ref, p16_ref, a2_ref,
             b2k_ref, m1632_ref, b_ref, or_ref, oi_ref, g2_ref):
    nv = y0b_ref.shape[0]
    dinv16 = dinv_ref[...]
    p = p_ref[...]
    m = m_ref[...]
    u = (0.5 * dinv16) * (p[0] + p[1] - m[0] - m[1])
    y0b = y0b_ref[...]
    xr = jnp.maximum(y0b - u, 0.0)
    xi = y0b + u
    dot = lambda a, b: jnp.dot(a, b, preferred_element_type=jnp.float32,
                   precision=lax.Precision.HIGHEST)
    zrw = dot(xr, k2_ref[...])
    ziw = dot(xi, k2_ref[...])
    b = b_ref[...]
    or_ref[...] = dot(zrw, p16_ref[...]) + b
    oi_ref[...] = dot(ziw, p16_ref[...]) + b
    dinv32 = dot(dinv16, m1632_ref[...])
    g2w = dinv32 * (dot(zrw, a2_ref[...]) + dot(ziw, b2k_ref[...]))
    g2_ref[...] = g2w.reshape(nv, 2, 128)


def _t3_body(or_ref, oi_ref, p_ref, m_ref, dinv_ref, m1632_ref, er_ref,
             ei_ref, k3_ref, po_ref, go_ref, b_ref, o3_ref, g3_ref):
    nv = or_ref.shape[0]
    dot = lambda a, b: jnp.dot(a, b, preferred_element_type=jnp.float32,
                   precision=lax.Precision.HIGHEST)
    dinv16 = dinv_ref[...]
    dinv32 = dot(dinv16, m1632_ref[...])
    p = p_ref[...].reshape(2, nv, 2, 128).reshape(2, nv, 256)
    m = m_ref[...].reshape(2, nv, 2, 128).reshape(2, nv, 256)
    u = (0.5 * dinv32) * (p[0] + p[1] - m[0] - m[1])
    u_r = dot(u, er_ref[...])
    u_i = dot(u, ei_ref[...])
    xr = jnp.maximum(or_ref[...] - u_i, 0.0)
    xi = oi_ref[...] + u_r
    zrw = dot(xr, k3_ref[...])
    ziw = dot(xi, k3_ref[...])
    po = po_ref[...]
    go = go_ref[...]
    o3_ref[...] = dot(zrw, po[0]) + dot(ziw, po[1]) + b_ref[...]
    g3_ref[...] = dinv16 * (dot(zrw, go[0]) + dot(ziw, go[1]))


def _t4_body(o3_ref, p_ref, m_ref, dinv_ref, bsw_ref, kr_ref, br_ref,
             out_ref):
    dot = lambda a, b: jnp.dot(a, b, preferred_element_type=jnp.float32,
                   precision=lax.Precision.HIGHEST)
    dinv16 = dinv_ref[...]
    p = p_ref[...]
    m = m_ref[...]
    u = (0.5 * dinv16) * (p[0] + p[1] - m[0] - m[1])
    h = o3_ref[...] + dot(u, bsw_ref[...])
    out_ref[...] = dot(h, kr_ref[...]) + br_ref[0, 0]


def _tc_call(body, out_shapes, *args):
    return pl.pallas_call(
        body, out_shape=out_shapes,
        compiler_params=pltpu.CompilerParams(
            vmem_limit_bytes=100 * 1024 * 1024))(*args)


def kernel(data_x, data_edge_index, W1, b1, W2, b2, W3, b3, Wr, br):
    n, _ = data_x.shape
    e = data_edge_index.shape[1]
    e_pad = _pad_blocks(e)
    n_pad = _pad_nodes(n)

    src = data_edge_index[0]
    dst = data_edge_index[1]
    pad = e_pad - e
    zpad = n + lax.iota(jnp.int32, pad) % (n_pad - n)
    src3 = jnp.concatenate([src, zpad]).reshape(e_pad // BLK, BLK)
    dst3 = jnp.concatenate([dst, zpad]).reshape(e_pad // BLK, BLK)

    zz16 = jnp.zeros((n_pad, 16), jnp.float32)
    zz32 = jnp.zeros((n_pad, 32), jnp.float32)
    zza = jnp.zeros((n_pad + 16 * NW, 16), jnp.float32)
    htab = jnp.zeros((BLK, 16), jnp.float32).at[:, 0].set(0.5)
    ntab = jnp.zeros((BLK, 16), jnp.float32).at[:, 0].set(-1.0)

    f32 = jnp.float32
    w1cat = jnp.concatenate([W1[0], W1[1]], axis=1)
    w2cat = jnp.concatenate([W2[0], W2[1]], axis=1)
    w3cat = jnp.concatenate([W3[0], W3[1]], axis=1)
    eye8 = jnp.eye(8, dtype=f32)
    kron = jnp.kron
    a16 = jnp.arange(16)
    a8 = jnp.arange(8)
    wb0 = kron(eye8, w1cat[:, :16])
    wb1 = kron(eye8, w1cat[:, 16:])
    s16 = kron(eye8, jnp.zeros((16, 16), f32).at[0].set(1.0))
    k2 = kron(eye8, w2cat)
    p16 = kron(eye8, jnp.zeros((32, 16), f32).at[a16, a16].set(1.0))
    a2 = kron(eye8, jnp.zeros((32, 32), f32).at[16 + a16, a16].set(1.0))
    b2k = kron(eye8, jnp.zeros((32, 32), f32).at[16 + a16, 16 + a16].set(1.0))
    m1632 = kron(eye8, jnp.zeros((16, 32), f32).at[0].set(1.0))
    er = kron(eye8, jnp.zeros((32, 16), f32).at[a16, a16].set(1.0))
    ei = kron(eye8, jnp.zeros((32, 16), f32).at[16 + a16, a16].set(1.0))
    k3 = kron(eye8, w3cat)
    po = jnp.stack([
        kron(eye8, jnp.zeros((16, 16), f32).at[a8, a8].set(1.0)),
        kron(eye8, jnp.zeros((16, 16), f32).at[a8, 8 + a8].set(1.0))])
    go = jnp.stack([
        kron(eye8, jnp.zeros((16, 16), f32).at[8 + a8, a8].set(1.0)),
        kron(eye8, jnp.zeros((16, 16), f32).at[8 + a8, 8 + a8].set(1.0))])
    bsw = kron(eye8, jnp.zeros((16, 16), f32)
               .at[8 + a8, a8].set(-1.0).at[a8, 8 + a8].set(1.0))
    kr = kron(eye8, Wr)
    b1t = jnp.tile(b1, 8).reshape(1, 128)
    b2t = jnp.tile(b2, 8).reshape(1, 128)
    b3t = jnp.tile(jnp.concatenate([b3, b3]), 8).reshape(1, 128)
    brr = br.reshape(1, 1)

    nv = n_pad // 8

    def v16(a):
        return a.reshape(NC, nv, 128)

    degacc = _deg_kernel(n_pad, e_pad)(src3, dst3, zza, htab, ntab)

    y0b, g1, dinv = _tc_call(
        _t1_body,
        [jax.ShapeDtypeStruct((nv, 128), f32),
         jax.ShapeDtypeStruct((nv, 128), f32),
         jax.ShapeDtypeStruct((nv, 128), f32)],
        data_x, wb0, wb1, s16, b1t, v16(degacc))

    p1, m1 = _prop_kernel(n_pad, e_pad, 16)(g1.reshape(n_pad, 16),
                                            src3, dst3, zz16)

    o_r, o_i, g2 = _tc_call(
        _t2_body,
        [jax.ShapeDtypeStruct((nv, 128), f32),
         jax.ShapeDtypeStruct((nv, 128), f32),
         jax.ShapeDtypeStruct((nv, 2, 128), f32)],
        y0b, v16(p1), v16(m1), dinv, k2, p16, a2, b2k, m1632, b2t)

    p2, m2 = _prop_kernel(n_pad, e_pad, 32)(g2.reshape(n_pad, 32),
                                            src3, dst3, zz32)

    o3, g3 = _tc_call(
        _t3_body,
        [jax.ShapeDtypeStruct((nv, 128), f32),
         jax.ShapeDtypeStruct((nv, 128), f32)],
        o_r, o_i, p2.reshape(NC, n_pad // 4, 128),
        m2.reshape(NC, n_pad // 4, 128), dinv, m1632, er, ei, k3, po, go,
        b3t)

    p3, m3 = _prop_kernel(n_pad, e_pad, 16)(g3.reshape(n_pad, 16),
                                            src3, dst3, zz16)

    out = _tc_call(
        _t4_body,
        [jax.ShapeDtypeStruct((nv, 8), f32)],
        o3, v16(p3), v16(m3), dinv, bsw, kr, brr)

    return out[0].reshape(n_pad, 1)[:n]
